# baseline (speedup 1.0000x reference)
import functools

import jax
import jax.numpy as jnp
from jax import lax
from jax.experimental import pallas as pl
from jax.experimental.pallas import tpu as pltpu
from jax.experimental.pallas import tpu_sc as plsc

N_VOCAB = 1000000
N_TOKENS = 16384 * 20
D_MODEL = 64
NUM_WORKERS = 32
B_PER_W = N_TOKENS // NUM_WORKERS
CHUNK = 128
N_CHUNKS = B_PER_W // CHUNK
NBUF = 4
N_GROUPS = N_CHUNKS // NBUF

A_W = 512
SPLIT = 512000
A_GRID = SPLIT // A_W
TAIL_START = (N_VOCAB // A_W) * A_W
TAIL_BLK = (TAIL_START - SPLIT) // A_W


def _a_body(x0_ref, x1_ref, tail_ref, o_ref):
    pid = pl.program_id(0)
    eye = jnp.eye(D_MODEL, dtype=jnp.float32)
    dn = (((0,), (0,)), ((), ()))
    y0 = jax.lax.dot_general(x0_ref[...], eye, dn,
                             preferred_element_type=jnp.float32)
    x1 = jnp.where(pid == TAIL_BLK, tail_ref[...], x1_ref[...])
    y1 = jax.lax.dot_general(x1, eye, dn,
                             preferred_element_type=jnp.float32)
    o_ref[...] = jnp.concatenate([y0, y1], axis=1)


_table_transpose = pl.pallas_call(
    _a_body,
    grid=(A_GRID,),
    in_specs=[
        pl.BlockSpec((D_MODEL, A_W), lambda i: (0, i)),
        pl.BlockSpec((D_MODEL, A_W),
                     lambda i: (0, jnp.minimum(i + SPLIT // A_W,
                                               N_VOCAB // A_W - 1))),
        pl.BlockSpec((D_MODEL, A_W), lambda i: (0, 0)),
    ],
    out_specs=pl.BlockSpec((A_W, 128), lambda i: (i, 0)),
    out_shape=jax.ShapeDtypeStruct((SPLIT, 128), jnp.float32),
)

_mesh = plsc.VectorSubcoreMesh(core_axis_name="c", subcore_axis_name="s")


@functools.partial(
    pl.kernel,
    mesh=_mesh,
    out_type=jax.ShapeDtypeStruct((N_TOKENS, D_MODEL), jnp.float32),
    scratch_types=[
        pltpu.VMEM((N_CHUNKS, CHUNK), jnp.int32),
        pltpu.VMEM((NBUF, CHUNK, D_MODEL), jnp.float32),
        [pltpu.SemaphoreType.DMA] * NBUF,
        [pltpu.SemaphoreType.DMA] * NBUF,
    ],
    compiler_params=pltpu.CompilerParams(use_tc_tiling_on_sc=False),
)
def _embed_sc(ids_hbm, table_hbm, out_hbm, idx_v, rows_v, gsems, osems):
    wid = lax.axis_index("s") * 2 + lax.axis_index("c")
    base = wid * B_PER_W
    pltpu.sync_copy(ids_hbm.at[pl.ds(wid * N_CHUNKS, N_CHUNKS)], idx_v)

    def gather(c, b):
        pltpu.async_copy(table_hbm.at[idx_v.at[c]], rows_v.at[b], gsems[b])

    def gather_wait(c, b):
        pltpu.make_async_copy(table_hbm.at[idx_v.at[c]], rows_v.at[b],
                              gsems[b]).wait()

    def writeback(c, b):
        pltpu.async_copy(rows_v.at[b],
                         out_hbm.at[pl.ds(base + c * CHUNK, CHUNK)],
                         osems[b])

    def writeback_wait(c, b):
        pltpu.make_async_copy(rows_v.at[b],
                              out_hbm.at[pl.ds(base + c * CHUNK, CHUNK)],
                              osems[b]).wait()

    for b in range(NBUF):
        gather(b, b)

    def group(g, carry):
        for b in range(NBUF):
            c = g * NBUF + b
            gather_wait(c, b)
            writeback(c, b)
            writeback_wait(c, b)

            @pl.when(g < N_GROUPS - 1)
            def _():
                gather(c + NBUF, b)
        return carry

    lax.fori_loop(0, N_GROUPS, group, 0)

def kernel(ids, embedding):
    i = ids.astype(jnp.int32)
    j = jnp.where(i < SPLIT, 2 * i, 2 * i - (2 * SPLIT - 1))
    flat_ids = j.reshape(NUM_WORKERS * N_CHUNKS, CHUNK)
    t_t = jnp.transpose(embedding)
    tail = jnp.pad(t_t[:, TAIL_START:], ((0, 0), (0, A_W - (N_VOCAB - TAIL_START))))
    table_rm = _table_transpose(t_t, t_t, tail).reshape(2 * SPLIT, D_MODEL)
    rows = _embed_sc(flat_ids, table_rm)
    return rows.reshape(16384, 20, D_MODEL)


# speedup vs baseline: 1.0320x; 1.0320x over previous
import functools

import jax
import jax.numpy as jnp
from jax import lax
from jax.experimental import pallas as pl
from jax.experimental.pallas import tpu as pltpu
from jax.experimental.pallas import tpu_sc as plsc

N_VOCAB = 1000000
N_TOKENS = 16384 * 20
D_MODEL = 64
NUM_WORKERS = 32
B_PER_W = N_TOKENS // NUM_WORKERS
CHUNK = 128
N_CHUNKS = B_PER_W // CHUNK
NBUF = 4
N_GROUPS = N_CHUNKS // NBUF

A_W = 512
SPLIT = 512000
A_GRID = SPLIT // A_W
TAIL_START = (N_VOCAB // A_W) * A_W
TAIL_BLK = (TAIL_START - SPLIT) // A_W


def _a_body(x0_ref, x1_ref, o_ref):
    y0 = jnp.transpose(x0_ref[...])
    y1 = jnp.transpose(x1_ref[...])
    o_ref[...] = jnp.concatenate([y0, y1], axis=1)


_table_transpose = pl.pallas_call(
    _a_body,
    grid=(A_GRID,),
    in_specs=[
        pl.BlockSpec((D_MODEL, A_W), lambda i: (0, i)),
        pl.BlockSpec((D_MODEL, A_W),
                     lambda i: (0, jnp.minimum(i + SPLIT // A_W,
                                               N_VOCAB // A_W - 1))),
    ],
    out_specs=pl.BlockSpec((A_W, 128), lambda i: (i, 0)),
    out_shape=jax.ShapeDtypeStruct((SPLIT, 128), jnp.float32),
)


def _tail_body(x0_ref, tail_ref, _table_in, o_ref):
    o_ref[...] = jnp.concatenate(
        [jnp.transpose(x0_ref[...]), jnp.transpose(tail_ref[...])], axis=1)


_tail_fix = pl.pallas_call(
    _tail_body,
    grid=(1,),
    in_specs=[
        pl.BlockSpec((D_MODEL, A_W), lambda i: (0, TAIL_BLK)),
        pl.BlockSpec((D_MODEL, A_W), lambda i: (0, 0)),
        pl.BlockSpec((A_W, 128), lambda i: (TAIL_BLK, 0)),
    ],
    out_specs=pl.BlockSpec((A_W, 128), lambda i: (TAIL_BLK, 0)),
    out_shape=jax.ShapeDtypeStruct((SPLIT, 128), jnp.float32),
    input_output_aliases={2: 0},
)

_mesh = plsc.VectorSubcoreMesh(core_axis_name="c", subcore_axis_name="s")


@functools.partial(
    pl.kernel,
    mesh=_mesh,
    out_type=jax.ShapeDtypeStruct((N_TOKENS, D_MODEL), jnp.float32),
    scratch_types=[
        pltpu.VMEM((N_CHUNKS, CHUNK), jnp.int32),
        pltpu.VMEM((NBUF, CHUNK, D_MODEL), jnp.float32),
        [pltpu.SemaphoreType.DMA] * NBUF,
        [pltpu.SemaphoreType.DMA] * NBUF,
    ],
    compiler_params=pltpu.CompilerParams(use_tc_tiling_on_sc=False),
)
def _embed_sc(ids_hbm, table_hbm, out_hbm, idx_v, rows_v, gsems, osems):
    wid = lax.axis_index("s") * 2 + lax.axis_index("c")
    base = wid * B_PER_W
    pltpu.sync_copy(ids_hbm.at[pl.ds(wid * N_CHUNKS, N_CHUNKS)], idx_v)

    def gather(c, b):
        pltpu.async_copy(table_hbm.at[idx_v.at[c]], rows_v.at[b], gsems[b])

    def gather_wait(c, b):
        pltpu.make_async_copy(table_hbm.at[idx_v.at[c]], rows_v.at[b],
                              gsems[b]).wait()

    def writeback(c, b):
        pltpu.async_copy(rows_v.at[b],
                         out_hbm.at[pl.ds(base + c * CHUNK, CHUNK)],
                         osems[b])

    def writeback_wait(c, b):
        pltpu.make_async_copy(rows_v.at[b],
                              out_hbm.at[pl.ds(base + c * CHUNK, CHUNK)],
                              osems[b]).wait()

    for b in range(NBUF):
        gather(b, b)

    def group(g, carry):
        for b in range(NBUF):
            c = g * NBUF + b
            gather_wait(c, b)
            writeback(c, b)
            writeback_wait(c, b)

            @pl.when(g < N_GROUPS - 1)
            def _():
                gather(c + NBUF, b)
        return carry

    lax.fori_loop(0, N_GROUPS, group, 0)

def kernel(ids, embedding):
    i = ids.astype(jnp.int32)
    j = jnp.where(i < SPLIT, 2 * i, 2 * i - (2 * SPLIT - 1))
    flat_ids = j.reshape(NUM_WORKERS * N_CHUNKS, CHUNK)
    t_t = jnp.transpose(embedding)
    tail = jnp.pad(t_t[:, TAIL_START:],
                   ((0, 0), (0, A_W - (N_VOCAB - TAIL_START))))
    table_pre = _table_transpose(t_t, t_t)
    table_rm = _tail_fix(t_t, tail, table_pre).reshape(2 * SPLIT, D_MODEL)
    rows = _embed_sc(flat_ids, table_rm)
    return rows.reshape(16384, 20, D_MODEL)


# speedup vs baseline: 1.8328x; 1.7760x over previous
import functools

import jax
import jax.numpy as jnp
from jax import lax
from jax.experimental import pallas as pl
from jax.experimental.pallas import tpu as pltpu
from jax.experimental.pallas import tpu_sc as plsc

N_VOCAB = 1000000
N_TOKENS = 16384 * 20
D_MODEL = 64
NUM_WORKERS = 32
B_PER_W = N_TOKENS // NUM_WORKERS
CHUNK = 128
N_CHUNKS = B_PER_W // CHUNK
NBUF = 4
N_GROUPS = N_CHUNKS // NBUF

A_W = 4096
SPLIT = 512000
A_GRID = SPLIT // A_W
TAIL_START = (N_VOCAB // A_W) * A_W
TAIL_BLK = (TAIL_START - SPLIT) // A_W


def _a_body(x0_ref, x1_ref, o_ref):
    y0 = jnp.transpose(x0_ref[...])
    y1 = jnp.transpose(x1_ref[...])
    o_ref[...] = jnp.concatenate([y0, y1], axis=1)


_table_transpose = pl.pallas_call(
    _a_body,
    grid=(A_GRID,),
    in_specs=[
        pl.BlockSpec((D_MODEL, A_W), lambda i: (0, i)),
        pl.BlockSpec((D_MODEL, A_W),
                     lambda i: (0, jnp.minimum(i + SPLIT // A_W,
                                               N_VOCAB // A_W - 1))),
    ],
    out_specs=pl.BlockSpec((A_W, 128), lambda i: (i, 0)),
    out_shape=jax.ShapeDtypeStruct((SPLIT, 128), jnp.float32),
)


def _tail_body(x0_ref, tail_ref, _table_in, o_ref):
    o_ref[...] = jnp.concatenate(
        [jnp.transpose(x0_ref[...]), jnp.transpose(tail_ref[...])], axis=1)


_tail_fix = pl.pallas_call(
    _tail_body,
    grid=(1,),
    in_specs=[
        pl.BlockSpec((D_MODEL, A_W), lambda i: (0, TAIL_BLK)),
        pl.BlockSpec((D_MODEL, A_W), lambda i: (0, 0)),
        pl.BlockSpec((A_W, 128), lambda i: (TAIL_BLK, 0)),
    ],
    out_specs=pl.BlockSpec((A_W, 128), lambda i: (TAIL_BLK, 0)),
    out_shape=jax.ShapeDtypeStruct((SPLIT, 128), jnp.float32),
    input_output_aliases={2: 0},
)

_mesh = plsc.VectorSubcoreMesh(core_axis_name="c", subcore_axis_name="s")


@functools.partial(
    pl.kernel,
    mesh=_mesh,
    out_type=jax.ShapeDtypeStruct((N_TOKENS, D_MODEL), jnp.float32),
    scratch_types=[
        pltpu.VMEM((N_CHUNKS, CHUNK), jnp.int32),
        pltpu.VMEM((NBUF, CHUNK, D_MODEL), jnp.float32),
        [pltpu.SemaphoreType.DMA] * NBUF,
        [pltpu.SemaphoreType.DMA] * NBUF,
    ],
    compiler_params=pltpu.CompilerParams(use_tc_tiling_on_sc=False),
)
def _embed_sc(ids_hbm, table_hbm, out_hbm, idx_v, rows_v, gsems, osems):
    wid = lax.axis_index("s") * 2 + lax.axis_index("c")
    base = wid * B_PER_W
    pltpu.sync_copy(ids_hbm.at[pl.ds(wid * N_CHUNKS, N_CHUNKS)], idx_v)

    def gather(c, b):
        pltpu.async_copy(table_hbm.at[idx_v.at[c]], rows_v.at[b], gsems[b])

    def gather_wait(c, b):
        pltpu.make_async_copy(table_hbm.at[idx_v.at[c]], rows_v.at[b],
                              gsems[b]).wait()

    def writeback(c, b):
        pltpu.async_copy(rows_v.at[b],
                         out_hbm.at[pl.ds(base + c * CHUNK, CHUNK)],
                         osems[b])

    def writeback_wait(c, b):
        pltpu.make_async_copy(rows_v.at[b],
                              out_hbm.at[pl.ds(base + c * CHUNK, CHUNK)],
                              osems[b]).wait()

    for b in range(NBUF):
        gather(b, b)

    def group(g, carry):
        for b in range(NBUF):
            c = g * NBUF + b
            gather_wait(c, b)
            writeback(c, b)
            writeback_wait(c, b)

            @pl.when(g < N_GROUPS - 1)
            def _():
                gather(c + NBUF, b)
        return carry

    lax.fori_loop(0, N_GROUPS, group, 0)

def kernel(ids, embedding):
    i = ids.astype(jnp.int32)
    j = jnp.where(i < SPLIT, 2 * i, 2 * i - (2 * SPLIT - 1))
    flat_ids = j.reshape(NUM_WORKERS * N_CHUNKS, CHUNK)
    t_t = jnp.transpose(embedding)
    tail = jnp.pad(t_t[:, TAIL_START:],
                   ((0, 0), (0, A_W - (N_VOCAB - TAIL_START))))
    table_pre = _table_transpose(t_t, t_t)
    table_rm = _tail_fix(t_t, tail, table_pre).reshape(2 * SPLIT, D_MODEL)
    rows = _embed_sc(flat_ids, table_rm)
    return rows.reshape(16384, 20, D_MODEL)


# speedup vs baseline: 1.9567x; 1.0676x over previous
import functools

import jax
import jax.numpy as jnp
from jax import lax
from jax.experimental import pallas as pl
from jax.experimental.pallas import tpu as pltpu
from jax.experimental.pallas import tpu_sc as plsc

N_VOCAB = 1000000
N_TOKENS = 16384 * 20
D_MODEL = 64
NUM_WORKERS = 32
B_PER_W = N_TOKENS // NUM_WORKERS
CHUNK = 128
N_CHUNKS = B_PER_W // CHUNK
NBUF = 4
N_GROUPS = N_CHUNKS // NBUF

A_W = 10240
SPLIT = 512000
A_GRID = SPLIT // A_W
TAIL_START = (N_VOCAB // A_W) * A_W
TAIL_BLK = (TAIL_START - SPLIT) // A_W


def _a_body(x0_ref, x1_ref, o_ref):
    y0 = jnp.transpose(x0_ref[...])
    y1 = jnp.transpose(x1_ref[...])
    o_ref[...] = jnp.concatenate([y0, y1], axis=1)


_table_transpose = pl.pallas_call(
    _a_body,
    grid=(A_GRID,),
    in_specs=[
        pl.BlockSpec((D_MODEL, A_W), lambda i: (0, i)),
        pl.BlockSpec((D_MODEL, A_W),
                     lambda i: (0, jnp.minimum(i + SPLIT // A_W,
                                               N_VOCAB // A_W - 1))),
    ],
    out_specs=pl.BlockSpec((A_W, 128), lambda i: (i, 0)),
    out_shape=jax.ShapeDtypeStruct((SPLIT, 128), jnp.float32),
)


def _tail_body(x0_ref, tail_ref, _table_in, o_ref):
    o_ref[...] = jnp.concatenate(
        [jnp.transpose(x0_ref[...]), jnp.transpose(tail_ref[...])], axis=1)


_tail_fix = pl.pallas_call(
    _tail_body,
    grid=(1,),
    in_specs=[
        pl.BlockSpec((D_MODEL, A_W), lambda i: (0, TAIL_BLK)),
        pl.BlockSpec((D_MODEL, A_W), lambda i: (0, 0)),
        pl.BlockSpec((A_W, 128), lambda i: (TAIL_BLK, 0)),
    ],
    out_specs=pl.BlockSpec((A_W, 128), lambda i: (TAIL_BLK, 0)),
    out_shape=jax.ShapeDtypeStruct((SPLIT, 128), jnp.float32),
    input_output_aliases={2: 0},
)

_mesh = plsc.VectorSubcoreMesh(core_axis_name="c", subcore_axis_name="s")


@functools.partial(
    pl.kernel,
    mesh=_mesh,
    out_type=jax.ShapeDtypeStruct((N_TOKENS, D_MODEL), jnp.float32),
    scratch_types=[
        pltpu.VMEM((N_CHUNKS, CHUNK), jnp.int32),
        pltpu.VMEM((NBUF, CHUNK, D_MODEL), jnp.float32),
        [pltpu.SemaphoreType.DMA] * NBUF,
        [pltpu.SemaphoreType.DMA] * NBUF,
    ],
    compiler_params=pltpu.CompilerParams(use_tc_tiling_on_sc=False),
)
def _embed_sc(ids_hbm, table_hbm, out_hbm, idx_v, rows_v, gsems, osems):
    wid = lax.axis_index("s") * 2 + lax.axis_index("c")
    base = wid * B_PER_W
    pltpu.sync_copy(ids_hbm.at[pl.ds(wid * N_CHUNKS, N_CHUNKS)], idx_v)

    def gather(c, b):
        pltpu.async_copy(table_hbm.at[idx_v.at[c]], rows_v.at[b], gsems[b])

    def gather_wait(c, b):
        pltpu.make_async_copy(table_hbm.at[idx_v.at[c]], rows_v.at[b],
                              gsems[b]).wait()

    def writeback(c, b):
        pltpu.async_copy(rows_v.at[b],
                         out_hbm.at[pl.ds(base + c * CHUNK, CHUNK)],
                         osems[b])

    def writeback_wait(c, b):
        pltpu.make_async_copy(rows_v.at[b],
                              out_hbm.at[pl.ds(base + c * CHUNK, CHUNK)],
                              osems[b]).wait()

    for b in range(NBUF):
        gather(b, b)

    def group(g, carry):
        for b in range(NBUF):
            c = g * NBUF + b
            gather_wait(c, b)
            writeback(c, b)
            writeback_wait(c, b)

            @pl.when(g < N_GROUPS - 1)
            def _():
                gather(c + NBUF, b)
        return carry

    lax.fori_loop(0, N_GROUPS, group, 0)

def kernel(ids, embedding):
    i = ids.astype(jnp.int32)
    j = jnp.where(i < SPLIT, 2 * i, 2 * i - (2 * SPLIT - 1))
    flat_ids = j.reshape(NUM_WORKERS * N_CHUNKS, CHUNK)
    t_t = jnp.transpose(embedding)
    tail = jnp.pad(t_t[:, TAIL_START:],
                   ((0, 0), (0, A_W - (N_VOCAB - TAIL_START))))
    table_pre = _table_transpose(t_t, t_t)
    table_rm = _tail_fix(t_t, tail, table_pre).reshape(2 * SPLIT, D_MODEL)
    rows = _embed_sc(flat_ids, table_rm)
    return rows.reshape(16384, 20, D_MODEL)


# speedup vs baseline: 1.9594x; 1.0014x over previous
import functools

import jax
import jax.numpy as jnp
from jax import lax
from jax.experimental import pallas as pl
from jax.experimental.pallas import tpu as pltpu
from jax.experimental.pallas import tpu_sc as plsc

N_VOCAB = 1000000
N_TOKENS = 16384 * 20
D_MODEL = 64
NUM_WORKERS = 32
B_PER_W = N_TOKENS // NUM_WORKERS
CHUNK = 128
N_CHUNKS = B_PER_W // CHUNK
NBUF = 4
N_GROUPS = N_CHUNKS // NBUF

A_W = 12800
SPLIT = 512000
A_GRID = SPLIT // A_W
TAIL_START = (N_VOCAB // A_W) * A_W
TAIL_BLK = (TAIL_START - SPLIT) // A_W


def _a_body(x0_ref, x1_ref, o_ref):
    y0 = jnp.transpose(x0_ref[...])
    y1 = jnp.transpose(x1_ref[...])
    o_ref[...] = jnp.concatenate([y0, y1], axis=1)


_table_transpose = pl.pallas_call(
    _a_body,
    grid=(A_GRID,),
    in_specs=[
        pl.BlockSpec((D_MODEL, A_W), lambda i: (0, i)),
        pl.BlockSpec((D_MODEL, A_W),
                     lambda i: (0, jnp.minimum(i + SPLIT // A_W,
                                               N_VOCAB // A_W - 1))),
    ],
    out_specs=pl.BlockSpec((A_W, 128), lambda i: (i, 0)),
    out_shape=jax.ShapeDtypeStruct((SPLIT, 128), jnp.float32),
)


def _tail_body(x0_ref, tail_ref, _table_in, o_ref):
    o_ref[...] = jnp.concatenate(
        [jnp.transpose(x0_ref[...]), jnp.transpose(tail_ref[...])], axis=1)


_tail_fix = pl.pallas_call(
    _tail_body,
    grid=(1,),
    in_specs=[
        pl.BlockSpec((D_MODEL, A_W), lambda i: (0, TAIL_BLK)),
        pl.BlockSpec((D_MODEL, A_W), lambda i: (0, 0)),
        pl.BlockSpec((A_W, 128), lambda i: (TAIL_BLK, 0)),
    ],
    out_specs=pl.BlockSpec((A_W, 128), lambda i: (TAIL_BLK, 0)),
    out_shape=jax.ShapeDtypeStruct((SPLIT, 128), jnp.float32),
    input_output_aliases={2: 0},
)

_mesh = plsc.VectorSubcoreMesh(core_axis_name="c", subcore_axis_name="s")


@functools.partial(
    pl.kernel,
    mesh=_mesh,
    out_type=jax.ShapeDtypeStruct((N_TOKENS, D_MODEL), jnp.float32),
    scratch_types=[
        pltpu.VMEM((N_CHUNKS, CHUNK), jnp.int32),
        pltpu.VMEM((NBUF, CHUNK, D_MODEL), jnp.float32),
        [pltpu.SemaphoreType.DMA] * NBUF,
        [pltpu.SemaphoreType.DMA] * NBUF,
    ],
    compiler_params=pltpu.CompilerParams(use_tc_tiling_on_sc=False),
)
def _embed_sc(ids_hbm, table_hbm, out_hbm, idx_v, rows_v, gsems, osems):
    wid = lax.axis_index("s") * 2 + lax.axis_index("c")
    base = wid * B_PER_W
    pltpu.sync_copy(ids_hbm.at[pl.ds(wid * N_CHUNKS, N_CHUNKS)], idx_v)

    def gather(c, b):
        pltpu.async_copy(table_hbm.at[idx_v.at[c]], rows_v.at[b], gsems[b])

    def gather_wait(c, b):
        pltpu.make_async_copy(table_hbm.at[idx_v.at[c]], rows_v.at[b],
                              gsems[b]).wait()

    def writeback(c, b):
        pltpu.async_copy(rows_v.at[b],
                         out_hbm.at[pl.ds(base + c * CHUNK, CHUNK)],
                         osems[b])

    def writeback_wait(c, b):
        pltpu.make_async_copy(rows_v.at[b],
                              out_hbm.at[pl.ds(base + c * CHUNK, CHUNK)],
                              osems[b]).wait()

    for b in range(NBUF):
        gather(b, b)

    def group(g, carry):
        for b in range(NBUF):
            c = g * NBUF + b
            gather_wait(c, b)
            writeback(c, b)
            writeback_wait(c, b)

            @pl.when(g < N_GROUPS - 1)
            def _():
                gather(c + NBUF, b)
        return carry

    lax.fori_loop(0, N_GROUPS, group, 0)

def kernel(ids, embedding):
    i = ids.astype(jnp.int32)
    j = jnp.where(i < SPLIT, 2 * i, 2 * i - (2 * SPLIT - 1))
    flat_ids = j.reshape(NUM_WORKERS * N_CHUNKS, CHUNK)
    t_t = jnp.transpose(embedding)
    tail = jnp.pad(t_t[:, TAIL_START:],
                   ((0, 0), (0, A_W - (N_VOCAB - TAIL_START))))
    table_pre = _table_transpose(t_t, t_t)
    table_rm = _tail_fix(t_t, tail, table_pre).reshape(2 * SPLIT, D_MODEL)
    rows = _embed_sc(flat_ids, table_rm)
    return rows.reshape(16384, 20, D_MODEL)
